# Initial kernel scaffold; baseline (speedup 1.0000x reference)
import functools

import jax
import jax.numpy as jnp
from jax import lax
from jax.experimental import pallas as pl
from jax.experimental.pallas import tpu as pltpu
from jax.experimental.pallas import tpu_sc as plsc

CHUNK = 512
STREAM = 128
NBUF = 2


@functools.partial(jax.jit, static_argnames=("n_chunks", "b_per_w", "num_cores"))
def _sc_gather(table, idx_flat, *, n_chunks, b_per_w, num_cores):
    d = table.shape[1]
    n_total = idx_flat.shape[0]
    mesh = plsc.VectorSubcoreMesh(core_axis_name="c", subcore_axis_name="s")

    @functools.partial(
        pl.kernel,
        out_type=jax.ShapeDtypeStruct((n_total, d), jnp.float32),
        mesh=mesh,
        scratch_types=[
            pltpu.VMEM((NBUF, CHUNK), jnp.int32),
            pltpu.VMEM((NBUF, CHUNK, d), jnp.float32),
            pltpu.SemaphoreType.DMA((NBUF,)),
            pltpu.SemaphoreType.DMA((NBUF,)),
            pltpu.SemaphoreType.DMA((NBUF,)),
        ],
    )
    def k(table_hbm, idx_hbm, out_hbm, idx_v, rows_v, sem_i, sem_g, sem_o):
        wid = lax.axis_index("s") * num_cores + lax.axis_index("c")
        base = wid * b_per_w

        def idx_copy(c, b):
            return pltpu.make_async_copy(
                idx_hbm.at[pl.ds(base + c * CHUNK, CHUNK)],
                idx_v.at[b],
                sem_i.at[b],
            )

        def gather_copy(b, j):
            return pltpu.make_async_copy(
                table_hbm.at[idx_v.at[b, pl.ds(j * STREAM, STREAM)]],
                rows_v.at[b, pl.ds(j * STREAM, STREAM), :],
                sem_g.at[b],
            )

        def out_copy(c, b):
            return pltpu.make_async_copy(
                rows_v.at[b],
                out_hbm.at[pl.ds(base + c * CHUNK, CHUNK), :],
                sem_o.at[b],
            )

        idx_copy(0, 0).start()

        def step(i, _):
            c2 = 2 * i
            for b in range(NBUF):
                c = c2 + b

                @pl.when(c + 1 < n_chunks)
                def _():
                    idx_copy(c + 1, b ^ 1).start()

                idx_copy(c, b).wait()

                @pl.when(c2 >= 2)
                def _():
                    out_copy(c - NBUF, b).wait()

                for j in range(CHUNK // STREAM):
                    gather_copy(b, j).start()
                for j in range(CHUNK // STREAM):
                    gather_copy(b, j).wait()

                out_copy(c, b).start()
            return _

        lax.fori_loop(0, n_chunks // NBUF, step, None)

        for b in range(NBUF):
            out_copy(n_chunks - NBUF + b, b).wait()

    return k(table, idx_flat)


def kernel(time, time_encoding):
    b, t = time.shape
    d = time_encoding.shape[1]
    idx_flat = jnp.asarray(time, jnp.int32).reshape(b * t)

    info = plsc.get_sparse_core_info()
    num_workers = info.num_cores * info.num_subcores
    b_per_w = (b * t) // num_workers
    assert b_per_w * num_workers == b * t and b_per_w % CHUNK == 0

    out = _sc_gather(
        jnp.asarray(time_encoding, jnp.float32),
        idx_flat,
        n_chunks=b_per_w // CHUNK,
        b_per_w=b_per_w,
        num_cores=info.num_cores,
    )
    return out.reshape(b, t, d)


# speedup vs baseline: 4.9868x; 4.9868x over previous
import functools

import jax
import jax.numpy as jnp
from jax import lax
from jax.experimental import pallas as pl
from jax.experimental.pallas import tpu as pltpu
from jax.experimental.pallas import tpu_sc as plsc

CHUNK = 512
STREAM = 128
NBUF = 2


@functools.partial(jax.jit, static_argnames=("n_chunks", "b_per_w", "num_cores"))
def _sc_gather(table, idx_flat, *, n_chunks, b_per_w, num_cores):
    d = table.shape[1]
    n_total = idx_flat.shape[0]
    mesh = plsc.VectorSubcoreMesh(core_axis_name="c", subcore_axis_name="s")

    @functools.partial(
        pl.kernel,
        out_type=jax.ShapeDtypeStruct((n_total, d), jnp.float32),
        mesh=mesh,
        scratch_types=[
            pltpu.VMEM((NBUF, CHUNK), jnp.int32),
            pltpu.VMEM((NBUF, CHUNK, d), jnp.float32),
            pltpu.SemaphoreType.DMA((NBUF,)),
            pltpu.SemaphoreType.DMA((NBUF,)),
            pltpu.SemaphoreType.DMA((NBUF,)),
        ],
        compiler_params=pltpu.CompilerParams(use_tc_tiling_on_sc=False),
    )
    def k(table_hbm, idx_hbm, out_hbm, idx_v, rows_v, sem_i, sem_g, sem_o):
        wid = lax.axis_index("s") * num_cores + lax.axis_index("c")
        base = wid * b_per_w

        def idx_copy(c, b):
            return pltpu.make_async_copy(
                idx_hbm.at[pl.ds(base + c * CHUNK, CHUNK)],
                idx_v.at[b],
                sem_i.at[b],
            )

        def gather_copy(b, j):
            return pltpu.make_async_copy(
                table_hbm.at[idx_v.at[b, pl.ds(j * STREAM, STREAM)]],
                rows_v.at[b, pl.ds(j * STREAM, STREAM), :],
                sem_g.at[b],
            )

        def out_copy(c, b):
            return pltpu.make_async_copy(
                rows_v.at[b],
                out_hbm.at[pl.ds(base + c * CHUNK, CHUNK), :],
                sem_o.at[b],
            )

        idx_copy(0, 0).start()

        def step(i, _):
            c2 = 2 * i
            for b in range(NBUF):
                c = c2 + b

                @pl.when(c + 1 < n_chunks)
                def _():
                    idx_copy(c + 1, b ^ 1).start()

                idx_copy(c, b).wait()

                @pl.when(c2 >= 2)
                def _():
                    out_copy(c - NBUF, b).wait()

                for j in range(CHUNK // STREAM):
                    gather_copy(b, j).start()
                for j in range(CHUNK // STREAM):
                    gather_copy(b, j).wait()

                out_copy(c, b).start()
            return _

        lax.fori_loop(0, n_chunks // NBUF, step, None)

        for b in range(NBUF):
            out_copy(n_chunks - NBUF + b, b).wait()

    return k(table, idx_flat)


def kernel(time, time_encoding):
    b, t = time.shape
    d = time_encoding.shape[1]
    idx_flat = jnp.asarray(time, jnp.int32).reshape(b * t)

    info = plsc.get_sparse_core_info()
    num_workers = info.num_cores * info.num_subcores
    b_per_w = (b * t) // num_workers
    assert b_per_w * num_workers == b * t and b_per_w % CHUNK == 0

    out = _sc_gather(
        jnp.asarray(time_encoding, jnp.float32),
        idx_flat,
        n_chunks=b_per_w // CHUNK,
        b_per_w=b_per_w,
        num_cores=info.num_cores,
    )
    return out.reshape(b, t, d)


# speedup vs baseline: 5.8492x; 1.1729x over previous
import functools

import jax
import jax.numpy as jnp
from jax import lax
from jax.experimental import pallas as pl
from jax.experimental.pallas import tpu as pltpu
from jax.experimental.pallas import tpu_sc as plsc

CHUNK = 512
STREAM = 128
NBUF = 2


@functools.partial(jax.jit, static_argnames=("n_chunks", "b_per_w", "num_cores"))
def _sc_gather(table, idx_flat, *, n_chunks, b_per_w, num_cores):
    d = table.shape[1]
    n_total = idx_flat.shape[0]
    mesh = plsc.VectorSubcoreMesh(core_axis_name="c", subcore_axis_name="s")

    @functools.partial(
        pl.kernel,
        out_type=jax.ShapeDtypeStruct((n_total, d), jnp.float32),
        mesh=mesh,
        scratch_types=[
            pltpu.VMEM((NBUF, CHUNK), jnp.int32),
            pltpu.VMEM((NBUF, CHUNK, d), jnp.float32),
            pltpu.VMEM_SHARED(table.shape, jnp.float32),
            pltpu.SemaphoreType.DMA((NBUF,)),
            pltpu.SemaphoreType.DMA((NBUF,)),
            pltpu.SemaphoreType.DMA((NBUF,)),
        ],
        compiler_params=pltpu.CompilerParams(use_tc_tiling_on_sc=False),
    )
    def k(table_hbm, idx_hbm, out_hbm, idx_v, rows_v, table_sh, sem_i, sem_g, sem_o):
        wid = lax.axis_index("s") * num_cores + lax.axis_index("c")
        base = wid * b_per_w

        @pl.when(lax.axis_index("s") == 0)
        def _():
            pltpu.sync_copy(table_hbm, table_sh)

        plsc.subcore_barrier()

        def idx_copy(c, b):
            return pltpu.make_async_copy(
                idx_hbm.at[pl.ds(base + c * CHUNK, CHUNK)],
                idx_v.at[b],
                sem_i.at[b],
            )

        def gather_copy(b, j):
            return pltpu.make_async_copy(
                table_sh.at[idx_v.at[b, pl.ds(j * STREAM, STREAM)]],
                rows_v.at[b, pl.ds(j * STREAM, STREAM), :],
                sem_g.at[b],
            )

        def out_copy(c, b):
            return pltpu.make_async_copy(
                rows_v.at[b],
                out_hbm.at[pl.ds(base + c * CHUNK, CHUNK), :],
                sem_o.at[b],
            )

        idx_copy(0, 0).start()

        def step(i, _):
            c2 = 2 * i
            for b in range(NBUF):
                c = c2 + b

                @pl.when(c + 1 < n_chunks)
                def _():
                    idx_copy(c + 1, b ^ 1).start()

                idx_copy(c, b).wait()

                @pl.when(c2 >= 2)
                def _():
                    out_copy(c - NBUF, b).wait()

                for j in range(CHUNK // STREAM):
                    gather_copy(b, j).start()
                for j in range(CHUNK // STREAM):
                    gather_copy(b, j).wait()

                out_copy(c, b).start()
            return _

        lax.fori_loop(0, n_chunks // NBUF, step, None)

        for b in range(NBUF):
            out_copy(n_chunks - NBUF + b, b).wait()

    return k(table, idx_flat)


def kernel(time, time_encoding):
    b, t = time.shape
    d = time_encoding.shape[1]
    idx_flat = jnp.asarray(time, jnp.int32).reshape(b * t)

    info = plsc.get_sparse_core_info()
    num_workers = info.num_cores * info.num_subcores
    b_per_w = (b * t) // num_workers
    assert b_per_w * num_workers == b * t and b_per_w % CHUNK == 0

    out = _sc_gather(
        jnp.asarray(time_encoding, jnp.float32),
        idx_flat,
        n_chunks=b_per_w // CHUNK,
        b_per_w=b_per_w,
        num_cores=info.num_cores,
    )
    return out.reshape(b, t, d)
